# Initial kernel scaffold; baseline (speedup 1.0000x reference)
import jax
import jax.numpy as jnp
from jax.experimental import pallas as pl


def kernel(idx, table):
    raise NotImplementedError("write your pallas kernel here")


# speedup vs baseline: 1.8163x; 1.8163x over previous
import functools

import jax
import jax.numpy as jnp
from jax import lax
from jax.experimental import pallas as pl
from jax.experimental.pallas import tpu as pltpu
from jax.experimental.pallas import tpu_sc as plsc

VOCAB = 8192
DIM = 8192
NLOOKUP = 8192
NWORKER = 32
BPW = NLOOKUP // NWORKER
CHUNK = 8
NCHUNK = BPW // CHUNK

_mesh = plsc.VectorSubcoreMesh(core_axis_name="c", subcore_axis_name="s")


@functools.partial(
    pl.kernel,
    mesh=_mesh,
    out_type=jax.ShapeDtypeStruct((NLOOKUP, DIM), jnp.float32),
    scratch_types=[
        pltpu.VMEM((BPW,), jnp.int32),
        pltpu.VMEM((CHUNK, DIM), jnp.float32),
        pltpu.SemaphoreType.DMA,
    ],
)
def _gather(idx_hbm, table_hbm, out_hbm, idx_v, buf, sem):
    wid = lax.axis_index("s") * 2 + lax.axis_index("c")
    base = wid * BPW
    pltpu.sync_copy(idx_hbm.at[pl.ds(base, BPW)], idx_v)

    def body(c, carry):
        rows = base + c * CHUNK
        pltpu.async_copy(
            table_hbm.at[idx_v.at[pl.ds(c * CHUNK, CHUNK)]], buf, sem
        ).wait()
        pltpu.sync_copy(buf, out_hbm.at[pl.ds(rows, CHUNK)])
        return carry

    lax.fori_loop(0, NCHUNK, body, 0)


def kernel(idx, table):
    flat_idx = idx.reshape(-1).astype(jnp.int32)
    out = _gather(flat_idx, table)
    return out.reshape(idx.shape[0], idx.shape[1], DIM)


# speedup vs baseline: 1.9703x; 1.0848x over previous
import functools

import jax
import jax.numpy as jnp
from jax import lax
from jax.experimental import pallas as pl
from jax.experimental.pallas import tpu as pltpu
from jax.experimental.pallas import tpu_sc as plsc

VOCAB = 8192
DIM = 8192
HALF = DIM // 2
NLOOKUP = 8192
NWORKER = 32
BPW = NLOOKUP // NWORKER
CHUNK = 8
NSTEP = 2 * (BPW // CHUNK)

_mesh = plsc.VectorSubcoreMesh(core_axis_name="c", subcore_axis_name="s")


@functools.partial(
    pl.kernel,
    mesh=_mesh,
    out_type=jax.ShapeDtypeStruct((NLOOKUP, DIM), jnp.float32),
    scratch_types=[
        pltpu.VMEM((BPW,), jnp.int32),
        pltpu.VMEM((CHUNK, HALF), jnp.float32),
        pltpu.VMEM((CHUNK, HALF), jnp.float32),
        pltpu.SemaphoreType.DMA,
        pltpu.SemaphoreType.DMA,
        pltpu.SemaphoreType.DMA,
        pltpu.SemaphoreType.DMA,
    ],
)
def _gather(idx_hbm, table_hbm, out_hbm, idx_v, buf0, buf1,
            gsem0, gsem1, osem0, osem1):
    wid = lax.axis_index("s") * 2 + lax.axis_index("c")
    base = wid * BPW
    pltpu.sync_copy(idx_hbm.at[pl.ds(base, BPW)], idx_v)

    bufs = (buf0, buf1)
    gsems = (gsem0, gsem1)
    osems = (osem0, osem1)

    def start_gather(s, b):
        c = s // 2
        h = s % 2
        pltpu.async_copy(
            table_hbm.at[idx_v.at[pl.ds(c * CHUNK, CHUNK)],
                         pl.ds(h * HALF, HALF)],
            bufs[b], gsems[b],
        )

    def start_out(s, b):
        c = s // 2
        h = s % 2
        pltpu.async_copy(
            bufs[b],
            out_hbm.at[pl.ds(base + c * CHUNK, CHUNK), pl.ds(h * HALF, HALF)],
            osems[b],
        )

    def wait(sem):
        pltpu.make_async_copy(
            out_hbm.at[pl.ds(0, CHUNK), pl.ds(0, HALF)], bufs[0], sem
        ).wait()

    start_gather(0, 0)
    start_gather(1, 1)

    def body(k, carry):
        for b in range(2):
            s = 2 * k + b
            wait(gsems[b])
            start_out(s, b)
            wait(osems[b])
            start_gather(s + 2, b)
        return carry

    lax.fori_loop(0, NSTEP // 2 - 1, body, 0)

    for b in range(2):
        s = NSTEP - 2 + b
        wait(gsems[b])
        start_out(s, b)
        wait(osems[b])


def kernel(idx, table):
    flat_idx = idx.reshape(-1).astype(jnp.int32)
    out = _gather(flat_idx, table)
    return out.reshape(idx.shape[0], idx.shape[1], DIM)
